# Initial kernel scaffold; baseline (speedup 1.0000x reference)
import jax
import jax.numpy as jnp
from jax.experimental import pallas as pl


def kernel(xyz, new_xyz, features):
    raise NotImplementedError("write your pallas kernel here")


# speedup vs baseline: 14.7230x; 14.7230x over previous
import functools

import numpy as np
import jax
import jax.numpy as jnp
from jax import lax
from jax.experimental import pallas as pl
from jax.experimental.pallas import tpu as pltpu
from jax.experimental.pallas import tpu_sc as plsc

RADIUS_SQ = np.float32(0.2 * 0.2)
NSAMPLE = 32
NC, NS, L = 2, 16, 16


def _build(B, N, NP, C):
    assert B == 2 * NC
    MPW = NP // 8
    CPW = C // 8
    MCH = 256
    NV = N // L
    VPC = 16

    mesh = plsc.VectorSubcoreMesh(
        core_axis_name="c", subcore_axis_name="s",
        num_cores=NC, num_subcores=NS)

    @functools.partial(
        pl.kernel,
        out_type=jax.ShapeDtypeStruct((B, 3 + C, NP, NSAMPLE), jnp.float32),
        mesh=mesh,
        compiler_params=pltpu.CompilerParams(use_tc_tiling_on_sc=False,
                                             needs_layout_passes=False),
        scratch_types=dict(
            x_v=pltpu.VMEM((N,), jnp.float32),
            y_v=pltpu.VMEM((N,), jnp.float32),
            z_v=pltpu.VMEM((N,), jnp.float32),
            xb_v=pltpu.VMEM((N,), jnp.float32),
            yb_v=pltpu.VMEM((N,), jnp.float32),
            zb_v=pltpu.VMEM((N,), jnp.float32),
            pp_v=pltpu.VMEM((N,), jnp.float32),
            cen_v=pltpu.VMEM((3, MPW + L), jnp.float32),
            qq_v=pltpu.VMEM((MPW + L,), jnp.float32),
            cand_v=pltpu.VMEM((NSAMPLE + VPC * L + L,), jnp.int32),
            idx_v=pltpu.VMEM((MPW * NSAMPLE,), jnp.int32),
            ox_v=pltpu.VMEM((3, MPW, NSAMPLE), jnp.float32),
            idxb_v=pltpu.VMEM((NP * NSAMPLE,), jnp.int32),
            row_v=pltpu.VMEM((N,), jnp.float32),
            ob_v=pltpu.VMEM((MCH, NSAMPLE), jnp.float32),
            sh_idx=pltpu.VMEM_SHARED((2, NP * NSAMPLE), jnp.int32),
        ),
    )
    def qag(xyz_hbm, new_hbm, pp_hbm, qq_hbm, feat_hbm, out_hbm, *,
            x_v, y_v, z_v, xb_v, yb_v, zb_v, pp_v, cen_v, qq_v, cand_v,
            idx_v, ox_v, idxb_v, row_v, ob_v, sh_idx):
        cid = lax.axis_index("c")
        sid = lax.axis_index("s")
        bslot = sid // 8
        b = cid * 2 + bslot
        m0 = (sid % 8) * MPW

        pltpu.sync_copy(xyz_hbm.at[b, 0], x_v)
        pltpu.sync_copy(xyz_hbm.at[b, 1], y_v)
        pltpu.sync_copy(xyz_hbm.at[b, 2], z_v)
        pltpu.sync_copy(pp_hbm.at[b], pp_v)
        pltpu.sync_copy(new_hbm.at[b, :, pl.ds(m0, MPW)],
                        cen_v.at[:, pl.ds(0, MPW)])
        pltpu.sync_copy(qq_hbm.at[b, pl.ds(m0, MPW)], qq_v.at[pl.ds(0, MPW)])

        iota = lax.iota(jnp.int32, L)
        zeros16 = jnp.zeros((L,), jnp.int32)

        def _bf16r(v):
            t = plsc.bitcast(v, jnp.int32)
            r = (t + jnp.int32(0x7FFF) + ((t >> 16) & 1)) & jnp.int32(-65536)
            return plsc.bitcast(r, jnp.float32)

        def pre_round(v, carry):
            off = v * L
            xb_v[pl.ds(off, L)] = _bf16r(x_v[pl.ds(off, L)])
            yb_v[pl.ds(off, L)] = _bf16r(y_v[pl.ds(off, L)])
            zb_v[pl.ds(off, L)] = _bf16r(z_v[pl.ds(off, L)])
            return carry

        lax.fori_loop(0, NV, pre_round, 0)

        def per_center(m, carry):
            cxv = cen_v[0, pl.ds(m, L)]
            cyv = cen_v[1, pl.ds(m, L)]
            czv = cen_v[2, pl.ds(m, L)]
            qqv = qq_v[pl.ds(m, L)]
            cxs = jnp.full((L,), cxv[0], jnp.float32)
            cys = jnp.full((L,), cyv[0], jnp.float32)
            czs = jnp.full((L,), czv[0], jnp.float32)
            cxb = _bf16r(cxs)
            cyb = _bf16r(cys)
            czb = _bf16r(czs)
            qqs = jnp.full((L,), qqv[0], jnp.float32)

            def per_chunk(q, cnt):
                def active(cnt):
                    base = q * (VPC * L)
                    for j in range(VPC):
                        off = base + j * L
                        xv = xb_v[pl.ds(off, L)]
                        yv = yb_v[pl.ds(off, L)]
                        zv = zb_v[pl.ds(off, L)]
                        ppv = pp_v[pl.ds(off, L)]
                        p0 = xv * cxb
                        p1 = yv * cyb
                        p2 = zv * czb
                        dot = (p0 + p1) + p2
                        d2 = (qqs + ppv) - (dot + dot)
                        msk = d2 < RADIUS_SQ
                        plsc.store_compressed(cand_v.at[pl.ds(cnt, L)],
                                              iota + off, mask=msk)
                        pc = plsc.all_reduce_population_count(msk)
                        cnt = cnt + pc[0]
                    return cnt

                return lax.cond(cnt < NSAMPLE, active, lambda c: c, cnt)

            cnt = lax.fori_loop(0, NV // VPC, per_chunk, jnp.int32(0))

            c0v = cand_v[pl.ds(0, L)]
            c1v = cand_v[pl.ds(L, L)]
            first = jnp.full((L,), c0v[0], jnp.int32)
            cntv = jnp.full((L,), cnt, jnp.int32)
            nz = cntv > 0
            for h, cv in ((0, c0v), (1, c1v)):
                lane = iota + h * L
                sel = jnp.where(lane < cntv, cv, first)
                sel = jnp.where(nz, sel, zeros16)
                idx_v[pl.ds(m * NSAMPLE + h * L, L)] = sel
                ox_v[0, m, pl.ds(h * L, L)] = (
                    plsc.load_gather(x_v, [sel]) - cxs)
                ox_v[1, m, pl.ds(h * L, L)] = (
                    plsc.load_gather(y_v, [sel]) - cys)
                ox_v[2, m, pl.ds(h * L, L)] = (
                    plsc.load_gather(z_v, [sel]) - czs)
            return carry

        lax.fori_loop(0, MPW, per_center, 0)

        for d in range(3):
            pltpu.sync_copy(ox_v.at[d], out_hbm.at[b, d, pl.ds(m0, MPW)])
        pltpu.sync_copy(idx_v,
                        sh_idx.at[bslot, pl.ds(m0 * NSAMPLE, MPW * NSAMPLE)])
        plsc.subcore_barrier()

        pltpu.sync_copy(sh_idx.at[bslot], idxb_v)
        c0 = (sid % 8) * CPW
        MU = 4

        def per_chan(k, carry):
            ch = c0 + k
            pltpu.sync_copy(feat_hbm.at[b, ch], row_v)

            def per_chunk(q, carry2):
                mbase = q * MCH
                ibase = mbase * NSAMPLE

                def per_m(t, carry3):
                    ob = t * MU
                    io = ibase + ob * NSAMPLE
                    for u in range(MU):
                        for h in range(2):
                            iv = idxb_v[pl.ds(io + u * NSAMPLE + h * L, L)]
                            ob_v[ob + u, pl.ds(h * L, L)] = plsc.load_gather(
                                row_v, [iv])
                    return carry3

                lax.fori_loop(0, MCH // MU, per_m, 0)
                pltpu.sync_copy(ob_v,
                                out_hbm.at[b, 3 + ch, pl.ds(mbase, MCH)])
                return carry2

            lax.fori_loop(0, NP // MCH, per_chunk, 0)
            return carry

        lax.fori_loop(0, CPW, per_chan, 0)

    return qag


def kernel(xyz, new_xyz, features):
    B, N, _ = xyz.shape
    NP = new_xyz.shape[1]
    C = features.shape[1]
    xyz_soa = jnp.transpose(xyz, (0, 2, 1))
    new_soa = jnp.transpose(new_xyz, (0, 2, 1))
    pp = jnp.sum(xyz * xyz, axis=-1)
    qq = jnp.sum(new_xyz * new_xyz, axis=-1)
    return _build(B, N, NP, C)(xyz_soa, new_soa, pp, qq, features)


# speedup vs baseline: 19.2041x; 1.3044x over previous
import functools

import numpy as np
import jax
import jax.numpy as jnp
from jax import lax
from jax.experimental import pallas as pl
from jax.experimental.pallas import tpu as pltpu
from jax.experimental.pallas import tpu_sc as plsc

RADIUS_SQ = np.float32(0.2 * 0.2)
NSAMPLE = 32
NC, NS, L = 2, 16, 16


def _build(B, N, NP, C):
    assert B == 2 * NC
    MPW = NP // 8
    CPW = C // 8
    MCH = 128
    NV = N // L
    VPC = 16

    mesh = plsc.VectorSubcoreMesh(
        core_axis_name="c", subcore_axis_name="s",
        num_cores=NC, num_subcores=NS)

    @functools.partial(
        pl.kernel,
        out_type=jax.ShapeDtypeStruct((B, 3 + C, NP, NSAMPLE), jnp.float32),
        mesh=mesh,
        compiler_params=pltpu.CompilerParams(use_tc_tiling_on_sc=False,
                                             needs_layout_passes=False),
        scratch_types=dict(
            x_v=pltpu.VMEM((N,), jnp.float32),
            y_v=pltpu.VMEM((N,), jnp.float32),
            z_v=pltpu.VMEM((N,), jnp.float32),
            xb_v=pltpu.VMEM((N,), jnp.bfloat16),
            yb_v=pltpu.VMEM((N,), jnp.bfloat16),
            zb_v=pltpu.VMEM((N,), jnp.bfloat16),
            pp_v=pltpu.VMEM((N,), jnp.float32),
            cen_v=pltpu.VMEM((3, MPW + L), jnp.float32),
            qq_v=pltpu.VMEM((MPW + L,), jnp.float32),
            cand_v=pltpu.VMEM((NSAMPLE + VPC * L + L,), jnp.int32),
            idx_v=pltpu.VMEM((MPW * NSAMPLE,), jnp.int32),
            ox_v=pltpu.VMEM((3, MPW, NSAMPLE), jnp.float32),
            idxb_v=pltpu.VMEM((NP * NSAMPLE,), jnp.int32),
            row_v=pltpu.VMEM((N,), jnp.float32),
            row2_v=pltpu.VMEM((N,), jnp.float32),
            ob_v=pltpu.VMEM((MCH, NSAMPLE), jnp.float32),
            ob2_v=pltpu.VMEM((MCH, NSAMPLE), jnp.float32),
            sh_idx=pltpu.VMEM_SHARED((2, NP * NSAMPLE), jnp.int32),
            sem_in=pltpu.SemaphoreType.DMA,
            sem_r=pltpu.SemaphoreType.DMA,
            sem_o1=pltpu.SemaphoreType.DMA,
            sem_o2=pltpu.SemaphoreType.DMA,
            sem_ox=pltpu.SemaphoreType.DMA,
        ),
    )
    def qag(xyz_hbm, new_hbm, pp_hbm, qq_hbm, feat_hbm, out_hbm, *,
            x_v, y_v, z_v, xb_v, yb_v, zb_v, pp_v, cen_v, qq_v, cand_v,
            idx_v, ox_v, idxb_v, row_v, row2_v, ob_v, ob2_v, sh_idx,
            sem_in, sem_r, sem_o1, sem_o2, sem_ox):
        cid = lax.axis_index("c")
        sid = lax.axis_index("s")
        bslot = sid // 8
        b = cid * 2 + bslot
        m0 = (sid % 8) * MPW
        c0 = (sid % 8) * CPW

        din = [
            pltpu.async_copy(xyz_hbm.at[b, 0], x_v, sem_in),
            pltpu.async_copy(xyz_hbm.at[b, 1], y_v, sem_in),
            pltpu.async_copy(xyz_hbm.at[b, 2], z_v, sem_in),
            pltpu.async_copy(pp_hbm.at[b], pp_v, sem_in),
            pltpu.async_copy(new_hbm.at[b, :, pl.ds(m0, MPW)],
                             cen_v.at[:, pl.ds(0, MPW)], sem_in),
            pltpu.async_copy(qq_hbm.at[b, pl.ds(m0, MPW)],
                             qq_v.at[pl.ds(0, MPW)], sem_in),
        ]
        pltpu.async_copy(feat_hbm.at[b, c0], row_v, sem_r)
        for dsc in din:
            dsc.wait()

        iota = lax.iota(jnp.int32, L)
        zeros16 = jnp.zeros((L,), jnp.int32)

        def _bf16r(v):
            t = plsc.bitcast(v, jnp.int32)
            r = (t + jnp.int32(0x7FFF) + ((t >> 16) & 1)) & jnp.int32(-65536)
            return plsc.bitcast(r, jnp.float32)

        def pre_round(v, carry):
            off = v * (2 * L)
            for src_v, dst_v in ((x_v, xb_v), (y_v, yb_v), (z_v, zb_v)):
                a0 = _bf16r(src_v[pl.ds(off, L)])
                a1 = _bf16r(src_v[pl.ds(off + L, L)])
                dst_v[pl.ds(off, 2 * L)] = plsc.pack(
                    a0, a1, format=plsc.PackFormat.INTERLEAVED)
            return carry

        lax.fori_loop(0, NV // 2, pre_round, 0)

        def per_center(m, carry):
            cxv = cen_v[0, pl.ds(m, L)]
            cyv = cen_v[1, pl.ds(m, L)]
            czv = cen_v[2, pl.ds(m, L)]
            qqv = qq_v[pl.ds(m, L)]
            cxs = jnp.full((L,), cxv[0], jnp.float32)
            cys = jnp.full((L,), cyv[0], jnp.float32)
            czs = jnp.full((L,), czv[0], jnp.float32)
            cxb = _bf16r(cxs)
            cyb = _bf16r(cys)
            czb = _bf16r(czs)
            qqs = jnp.full((L,), qqv[0], jnp.float32)

            def per_chunk(q, cnt):
                def active(cnt):
                    base = q * (VPC * L)
                    for j in range(0, VPC, 2):
                        off = base + j * L
                        xs = plsc.unpack(xb_v[pl.ds(off, 2 * L)],
                                         format=plsc.PackFormat.INTERLEAVED)
                        ys = plsc.unpack(yb_v[pl.ds(off, 2 * L)],
                                         format=plsc.PackFormat.INTERLEAVED)
                        zs = plsc.unpack(zb_v[pl.ds(off, 2 * L)],
                                         format=plsc.PackFormat.INTERLEAVED)
                        for u in range(2):
                            offu = off + u * L
                            ppv = pp_v[pl.ds(offu, L)]
                            p0 = xs[u] * cxb
                            p1 = ys[u] * cyb
                            p2 = zs[u] * czb
                            dot = (p0 + p1) + p2
                            d2 = (qqs + ppv) - (dot + dot)
                            msk = d2 < RADIUS_SQ
                            plsc.store_compressed(cand_v.at[pl.ds(cnt, L)],
                                                  iota + offu, mask=msk)
                            pc = plsc.all_reduce_population_count(msk)
                            cnt = cnt + pc[0]
                    return cnt

                return lax.cond(cnt < NSAMPLE, active, lambda c: c, cnt)

            cnt = lax.fori_loop(0, NV // VPC, per_chunk, jnp.int32(0))

            c0v = cand_v[pl.ds(0, L)]
            c1v = cand_v[pl.ds(L, L)]
            first = jnp.full((L,), c0v[0], jnp.int32)
            cntv = jnp.full((L,), cnt, jnp.int32)
            nz = cntv > 0
            for h, cv in ((0, c0v), (1, c1v)):
                lane = iota + h * L
                sel = jnp.where(lane < cntv, cv, first)
                sel = jnp.where(nz, sel, zeros16)
                idx_v[pl.ds(m * NSAMPLE + h * L, L)] = sel
                ox_v[0, m, pl.ds(h * L, L)] = (
                    plsc.load_gather(x_v, [sel]) - cxs)
                ox_v[1, m, pl.ds(h * L, L)] = (
                    plsc.load_gather(y_v, [sel]) - cys)
                ox_v[2, m, pl.ds(h * L, L)] = (
                    plsc.load_gather(z_v, [sel]) - czs)
            return carry

        lax.fori_loop(0, MPW, per_center, 0)

        for d in range(3):
            pltpu.async_copy(ox_v.at[d], out_hbm.at[b, d, pl.ds(m0, MPW)],
                             sem_ox)
        pltpu.sync_copy(idx_v,
                        sh_idx.at[bslot, pl.ds(m0 * NSAMPLE, MPW * NSAMPLE)])
        plsc.subcore_barrier()

        pltpu.sync_copy(sh_idx.at[bslot], idxb_v)
        MU = 4
        NQ = NP // MCH
        obufs = (ob_v, ob2_v)
        osems = (sem_o1, sem_o2)

        def do_chan(ch, row):
            for q in range(NQ):
                obuf = obufs[q % 2]
                osem = osems[q % 2]
                mbase = q * MCH
                ibase = mbase * NSAMPLE
                dst = out_hbm.at[b, 3 + ch, pl.ds(mbase, MCH)]
                if q >= 2:
                    pltpu.make_async_copy(obuf, dst, osem).wait()
                else:
                    @pl.when(ch > c0)
                    def _():
                        pltpu.make_async_copy(obuf, dst, osem).wait()

                def per_m(t, carry3, obuf=obuf, ibase=ibase):
                    ob = t * MU
                    io = ibase + ob * NSAMPLE
                    for u in range(MU):
                        for h in range(2):
                            iv = idxb_v[pl.ds(io + u * NSAMPLE + h * L, L)]
                            obuf[ob + u, pl.ds(h * L, L)] = plsc.load_gather(
                                row, [iv])
                    return carry3

                lax.fori_loop(0, MCH // MU, per_m, 0)
                pltpu.async_copy(obuf, dst, osem)

        def per_pair(k2, carry):
            ch0 = c0 + 2 * k2
            ch1 = ch0 + 1
            pltpu.make_async_copy(feat_hbm.at[b, ch0], row_v, sem_r).wait()
            pltpu.async_copy(feat_hbm.at[b, ch1], row2_v, sem_r)
            do_chan(ch0, row_v)
            pltpu.make_async_copy(feat_hbm.at[b, ch1], row2_v, sem_r).wait()
            ch2 = jnp.minimum(ch0 + 2, c0 + CPW - 1)
            pltpu.async_copy(feat_hbm.at[b, ch2], row_v, sem_r)
            do_chan(ch1, row2_v)
            return carry

        lax.fori_loop(0, CPW // 2, per_pair, 0)

        chl = c0 + CPW - 1
        pltpu.make_async_copy(feat_hbm.at[b, chl], row_v, sem_r).wait()
        pltpu.make_async_copy(
            ob_v, out_hbm.at[b, 3 + chl, pl.ds(2 * MCH, MCH)], sem_o1).wait()
        pltpu.make_async_copy(
            ob2_v, out_hbm.at[b, 3 + chl, pl.ds(3 * MCH, MCH)], sem_o2).wait()
        for d in range(3):
            pltpu.make_async_copy(ox_v.at[d],
                                  out_hbm.at[b, d, pl.ds(m0, MPW)],
                                  sem_ox).wait()

    return qag


def kernel(xyz, new_xyz, features):
    B, N, _ = xyz.shape
    NP = new_xyz.shape[1]
    C = features.shape[1]
    xyz_soa = jnp.transpose(xyz, (0, 2, 1))
    new_soa = jnp.transpose(new_xyz, (0, 2, 1))
    pp = jnp.sum(xyz * xyz, axis=-1)
    qq = jnp.sum(new_xyz * new_xyz, axis=-1)
    return _build(B, N, NP, C)(xyz_soa, new_soa, pp, qq, features)
